# Initial kernel scaffold; baseline (speedup 1.0000x reference)
import jax
import jax.numpy as jnp
from jax.experimental import pallas as pl


def kernel(hidden_states, attention_mask, centers, Wq, bq, Wk, bk, Wv, bv, Wo, bo, ln1_g, ln1_b, W1, b1, W2, b2, ln2_g, ln2_b):
    raise NotImplementedError("write your pallas kernel here")


# speedup vs baseline: 12.7340x; 12.7340x over previous
import jax
import jax.numpy as jnp
from jax.experimental import pallas as pl
from jax.experimental.pallas import tpu as pltpu

E = 8
B, S, D = 4, 2048, 768
H, DH = 12, 64
DFF = 3072
EPS = 1e-12

QKV_SCHUNK = 1024
QCHUNK = 512
FFN_SCHUNK = 1024
FCHUNK = 1536

_VMEM_LIM = 63 * 1024 * 1024


def _routing_kernel(x_ref, c_ref, out_ref):
    m = jnp.mean(x_ref[...], axis=1)
    diff = m[:, None, :] - c_ref[...][None, :, :]
    d2 = jnp.sum(diff * diff, axis=-1)
    ids = jax.lax.broadcasted_iota(jnp.int32, (B, E), 1)
    mn = jnp.min(d2, axis=1, keepdims=True)
    idx = jnp.min(jnp.where(d2 == mn, ids, E), axis=1, keepdims=True)
    out_ref[...] = idx


def _qkv_kernel(idx_ref, x_ref, wq_ref, wk_ref, wv_ref,
                bq_ref, bk_ref, bv_ref, q_ref, k_ref, v_ref):
    e = idx_ref[pl.program_id(0)]
    x = x_ref[0]
    q_ref[0] = jnp.dot(x, wq_ref[0], preferred_element_type=jnp.float32) + bq_ref[e]
    k_ref[0] = jnp.dot(x, wk_ref[0], preferred_element_type=jnp.float32) + bk_ref[e]
    v_ref[0] = jnp.dot(x, wv_ref[0], preferred_element_type=jnp.float32) + bv_ref[e]


def _attn_kernel(idx_ref, q_ref, k_ref, v_ref, x_ref, mask_ref, wo_ref,
                 bo_ref, g1_ref, be1_ref, out_ref, ctx_ref):
    e = idx_ref[pl.program_id(0)]
    ext = (1.0 - mask_ref[0]) * -10000.0
    for h in range(H):
        sl = slice(h * DH, (h + 1) * DH)
        qh = q_ref[0][:, sl] * 0.125
        kh = k_ref[0][:, sl]
        s = jax.lax.dot_general(qh, kh, (((1,), (1,)), ((), ())),
                                preferred_element_type=jnp.float32)
        s = s + ext
        mx = jnp.max(s, axis=1, keepdims=True)
        p = jnp.exp(s - mx)
        p = p / jnp.sum(p, axis=1, keepdims=True)
        ctx_ref[:, sl] = jnp.dot(p, v_ref[0][:, sl],
                                 preferred_element_type=jnp.float32)
    o = x_ref[0] + jnp.dot(ctx_ref[...], wo_ref[0],
                           preferred_element_type=jnp.float32) + bo_ref[e]
    mu = jnp.mean(o, axis=-1, keepdims=True)
    var = jnp.mean((o - mu) ** 2, axis=-1, keepdims=True)
    out_ref[0] = (o - mu) / jnp.sqrt(var + EPS) * g1_ref[e] + be1_ref[e]


def _ffn_kernel(idx_ref, x_ref, w1_ref, w2_ref, b1_ref, b2_ref,
                g2_ref, be2_ref, out_ref, acc_ref):
    e = idx_ref[pl.program_id(0)]
    c = pl.program_id(2)
    x = x_ref[0]
    h1 = jax.nn.gelu(jnp.dot(x, w1_ref[0], preferred_element_type=jnp.float32)
                     + b1_ref[e])
    y = jnp.dot(h1, w2_ref[0], preferred_element_type=jnp.float32)

    @pl.when(c == 0)
    def _():
        acc_ref[...] = y

    @pl.when(c > 0)
    def _():
        acc_ref[...] += y

    @pl.when(c == DFF // FCHUNK - 1)
    def _():
        o = x + acc_ref[...] + b2_ref[e]
        mu = jnp.mean(o, axis=-1, keepdims=True)
        var = jnp.mean((o - mu) ** 2, axis=-1, keepdims=True)
        out_ref[0] = (o - mu) / jnp.sqrt(var + EPS) * g2_ref[e] + be2_ref[e]


def kernel(hidden_states, attention_mask, centers, Wq, bq, Wk, bk, Wv, bv,
           Wo, bo, ln1_g, ln1_b, W1, b1, W2, b2, ln2_g, ln2_b):
    min_idx = pl.pallas_call(
        _routing_kernel,
        out_shape=jax.ShapeDtypeStruct((B, 1), jnp.int32),
    )(hidden_states, centers).reshape(B)

    mask3 = attention_mask.reshape(B, 1, S)

    def _full(shape, ngrid):
        nd = len(shape)
        return pl.BlockSpec(shape, lambda *a, _n=nd: (0,) * _n)

    qkv_shape = jax.ShapeDtypeStruct((B, S, D), jnp.float32)
    q, k, v = pl.pallas_call(
        _qkv_kernel,
        grid_spec=pltpu.PrefetchScalarGridSpec(
            num_scalar_prefetch=1,
            grid=(B, S // QKV_SCHUNK),
            in_specs=[
                pl.BlockSpec((1, QKV_SCHUNK, D), lambda b, s, idx: (b, s, 0)),
                pl.BlockSpec((1, D, D), lambda b, s, idx: (idx[b], 0, 0)),
                pl.BlockSpec((1, D, D), lambda b, s, idx: (idx[b], 0, 0)),
                pl.BlockSpec((1, D, D), lambda b, s, idx: (idx[b], 0, 0)),
                _full((E, D), 2), _full((E, D), 2), _full((E, D), 2),
            ],
            out_specs=[
                pl.BlockSpec((1, QKV_SCHUNK, D), lambda b, s, idx: (b, s, 0)),
                pl.BlockSpec((1, QKV_SCHUNK, D), lambda b, s, idx: (b, s, 0)),
                pl.BlockSpec((1, QKV_SCHUNK, D), lambda b, s, idx: (b, s, 0)),
            ],
        ),
        out_shape=[qkv_shape, qkv_shape, qkv_shape],
        compiler_params=pltpu.CompilerParams(vmem_limit_bytes=_VMEM_LIM),
    )(min_idx, hidden_states, Wq, Wk, Wv, bq, bk, bv)

    attn = pl.pallas_call(
        _attn_kernel,
        grid_spec=pltpu.PrefetchScalarGridSpec(
            num_scalar_prefetch=1,
            grid=(B, S // QCHUNK),
            in_specs=[
                pl.BlockSpec((1, QCHUNK, D), lambda b, qi, idx: (b, qi, 0)),
                pl.BlockSpec((1, S, D), lambda b, qi, idx: (b, 0, 0)),
                pl.BlockSpec((1, S, D), lambda b, qi, idx: (b, 0, 0)),
                pl.BlockSpec((1, QCHUNK, D), lambda b, qi, idx: (b, qi, 0)),
                pl.BlockSpec((1, 1, S), lambda b, qi, idx: (b, 0, 0)),
                pl.BlockSpec((1, D, D), lambda b, qi, idx: (idx[b], 0, 0)),
                _full((E, D), 2), _full((E, D), 2), _full((E, D), 2),
            ],
            out_specs=pl.BlockSpec((1, QCHUNK, D), lambda b, qi, idx: (b, qi, 0)),
            scratch_shapes=[pltpu.VMEM((QCHUNK, D), jnp.float32)],
        ),
        out_shape=jax.ShapeDtypeStruct((B, S, D), jnp.float32),
        compiler_params=pltpu.CompilerParams(vmem_limit_bytes=_VMEM_LIM),
    )(min_idx, q, k, v, hidden_states, mask3, Wo, bo, ln1_g, ln1_b)

    out = pl.pallas_call(
        _ffn_kernel,
        grid_spec=pltpu.PrefetchScalarGridSpec(
            num_scalar_prefetch=1,
            grid=(B, S // FFN_SCHUNK, DFF // FCHUNK),
            in_specs=[
                pl.BlockSpec((1, FFN_SCHUNK, D), lambda b, s, c, idx: (b, s, 0)),
                pl.BlockSpec((1, D, FCHUNK), lambda b, s, c, idx: (idx[b], 0, c)),
                pl.BlockSpec((1, FCHUNK, D), lambda b, s, c, idx: (idx[b], c, 0)),
                pl.BlockSpec((E, FCHUNK), lambda b, s, c, idx: (0, c)),
                _full((E, D), 3), _full((E, D), 3), _full((E, D), 3),
            ],
            out_specs=pl.BlockSpec((1, FFN_SCHUNK, D), lambda b, s, c, idx: (b, s, 0)),
            scratch_shapes=[pltpu.VMEM((FFN_SCHUNK, D), jnp.float32)],
        ),
        out_shape=jax.ShapeDtypeStruct((B, S, D), jnp.float32),
        compiler_params=pltpu.CompilerParams(vmem_limit_bytes=_VMEM_LIM),
    )(min_idx, attn, W1, W2, b1, b2, ln2_g, ln2_b)
    return out
